# baseline (speedup 1.0000x reference)
import jax
import jax.numpy as jnp
from jax import lax
from jax.experimental import pallas as pl
from jax.experimental.pallas import tpu as pltpu
from jax.experimental.pallas import tpu_sc as plsc

NUM_CORES = 2
NUM_SUBCORES = 16
LANES = 16
NUM_WORKERS = NUM_CORES * NUM_SUBCORES

BSZ = 4
SEQ_LEN = 4096
EMBED_DIM = 1024
TOTAL_ROWS = BSZ * SEQ_LEN
POS_PER_WORKER = SEQ_LEN // NUM_WORKERS
POS_CHUNK = 32
NUM_CHUNKS = POS_PER_WORKER // POS_CHUNK
VREGS_PER_SLICE = POS_PER_WORKER // LANES
ROW = EMBED_DIM
CHUNK_F = POS_CHUNK * ROW


def _lane_shuffle(x, idx):
    return lax.gather(
        x, idx[:, None],
        lax.GatherDimensionNumbers(
            offset_dims=(), collapsed_slice_dims=(0,), start_index_map=(0,)),
        (1,), mode=lax.GatherScatterMode.PROMISE_IN_BOUNDS)


def _sc_body(inp_hbm, w_hbm, out_hbm, inp_v, zrow, rows0, rows1,
             gsem, ssem0, ssem1, fsem):
    wid = lax.axis_index("s") * NUM_CORES + lax.axis_index("c")
    pos0 = wid * POS_PER_WORKER

    bufs = (rows0, rows1)
    ssems = (ssem0, ssem1)
    lane = lax.iota(jnp.int32, LANES)

    tok_copies = [
        pltpu.async_copy(
            inp_hbm.at[pl.ds(b * SEQ_LEN + pos0, POS_PER_WORKER)],
            inp_v.at[pl.ds(b * POS_PER_WORKER, POS_PER_WORKER)], fsem)
        for b in range(BSZ)
    ]
    zero16 = jnp.zeros((LANES,), jnp.float32)
    for j in range(ROW // LANES):
        zrow[pl.ds(j * LANES, LANES)] = zero16

    def drain_scatters(c):
        for _ in range(BSZ):
            pltpu.make_async_copy(
                out_hbm.at[pl.ds(0, CHUNK_F)],
                bufs[c % 2],
                ssems[c % 2]).wait()

    def stage_in(c):
        return pltpu.async_copy(
            w_hbm.at[pl.ds((pos0 + c * POS_CHUNK + 1) * ROW, CHUNK_F)],
            bufs[c % 2], gsem)

    ghs = [stage_in(0), stage_in(1)]
    for c in range(NUM_CHUNKS):
        buf = bufs[c % 2]
        ssem = ssems[c % 2]
        ghs[c].wait()
        for b in range(BSZ):
            pltpu.async_copy(
                buf,
                out_hbm.at[pl.ds((b * SEQ_LEN + pos0 + c * POS_CHUNK) * ROW,
                                 CHUNK_F)],
                ssem)
        if c + 2 < NUM_CHUNKS:
            drain_scatters(c)
            ghs.append(stage_in(c + 2))
    drain_scatters(NUM_CHUNKS - 2)
    drain_scatters(NUM_CHUNKS - 1)
    for h in tok_copies:
        h.wait()

    def any_true(mask_i32):
        m = mask_i32
        for sh in (1, 2, 4, 8):
            m = m | _lane_shuffle(m, lane ^ sh)
        return m[0] > 0

    toks = [
        inp_v[pl.ds(b * POS_PER_WORKER + v * LANES, LANES)]
        for b in range(BSZ) for v in range(VREGS_PER_SLICE)
    ]
    worker_acc = toks[0] == 0
    for t in toks[1:]:
        worker_acc = worker_acc | (t == 0)

    @pl.when(any_true(jnp.where(worker_acc, 1, 0)))
    def _fix_worker():
        for b in range(BSZ):
            for v in range(VREGS_PER_SLICE):
                tok = inp_v[pl.ds(b * POS_PER_WORKER + v * LANES, LANES)]
                pad = jnp.where(tok == 0, 1, 0)

                @pl.when(any_true(pad))
                def _fix_group(pad=pad, b=b, v=v):
                    row0 = b * SEQ_LEN + pos0 + v * LANES
                    for r in range(LANES):
                        is_pad = _lane_shuffle(pad, (lane + r) & (LANES - 1))

                        @pl.when(is_pad[0] > 0)
                        def _fix_row(r=r, row0=row0):
                            pltpu.sync_copy(
                                zrow,
                                out_hbm.at[pl.ds((row0 + r) * ROW, ROW)])


@jax.jit
def _sc_embed(inp_flat, w_flat):
    mesh = plsc.VectorSubcoreMesh(core_axis_name="c", subcore_axis_name="s")
    k = pl.kernel(
        _sc_body,
        out_type=jax.ShapeDtypeStruct((TOTAL_ROWS * ROW,), jnp.float32),
        mesh=mesh,
        scratch_types=[
            pltpu.VMEM((BSZ * POS_PER_WORKER,), jnp.int32),
            pltpu.VMEM((ROW,), jnp.float32),
            pltpu.VMEM((CHUNK_F,), jnp.float32),
            pltpu.VMEM((CHUNK_F,), jnp.float32),
            pltpu.SemaphoreType.DMA,
            pltpu.SemaphoreType.DMA,
            pltpu.SemaphoreType.DMA,
            pltpu.SemaphoreType.DMA,
        ],
    )
    return k(inp_flat, w_flat)


def kernel(input, weights):
    inp_flat = input.reshape(-1)
    out = _sc_embed(inp_flat, weights.reshape(-1))
    return out.reshape(BSZ, SEQ_LEN, EMBED_DIM)


# speedup vs baseline: 3.1693x; 3.1693x over previous
import jax
import jax.numpy as jnp
from jax import lax
from jax.experimental import pallas as pl
from jax.experimental.pallas import tpu as pltpu
from jax.experimental.pallas import tpu_sc as plsc

NUM_CORES = 2
NUM_SUBCORES = 16
LANES = 16
NUM_WORKERS = NUM_CORES * NUM_SUBCORES

BSZ = 4
SEQ_LEN = 4096
EMBED_DIM = 1024
TOTAL_ROWS = BSZ * SEQ_LEN
POS_PER_WORKER = SEQ_LEN // NUM_WORKERS
POS_CHUNK = 32
NUM_CHUNKS = POS_PER_WORKER // POS_CHUNK
VREGS_PER_SLICE = POS_PER_WORKER // LANES


def _lane_shuffle(x, idx):
    return lax.gather(
        x, idx[:, None],
        lax.GatherDimensionNumbers(
            offset_dims=(), collapsed_slice_dims=(0,), start_index_map=(0,)),
        (1,), mode=lax.GatherScatterMode.PROMISE_IN_BOUNDS)


def _sc_body(inp_hbm, w_hbm, out_hbm, inp_v, idx_v, fix_idx, fixbuf,
             rows0, rows1, gsem, ssem0, ssem1, fsem):
    wid = lax.axis_index("s") * NUM_CORES + lax.axis_index("c")
    pos0 = wid * POS_PER_WORKER

    bufs = (rows0, rows1)
    ssems = (ssem0, ssem1)
    lane = lax.iota(jnp.int32, LANES)

    for v in range(POS_PER_WORKER // LANES):
        idx_v[pl.ds(v * LANES, LANES)] = lane + (pos0 + v * LANES + 1)

    tok_copies = [
        pltpu.async_copy(
            inp_hbm.at[pl.ds(b * SEQ_LEN + pos0, POS_PER_WORKER)],
            inp_v.at[pl.ds(b * POS_PER_WORKER, POS_PER_WORKER)], fsem)
        for b in range(BSZ)
    ]

    def drain_scatters(c):
        for _ in range(BSZ):
            pltpu.make_async_copy(
                out_hbm.at[pl.ds(0, POS_CHUNK)],
                bufs[c % 2],
                ssems[c % 2]).wait()

    def stage_in(c):
        return pltpu.async_copy(
            w_hbm.at[idx_v.at[pl.ds(c * POS_CHUNK, POS_CHUNK)]],
            bufs[c % 2], gsem)

    ghs = [stage_in(0), stage_in(1)]
    for c in range(NUM_CHUNKS):
        buf = bufs[c % 2]
        ssem = ssems[c % 2]
        ghs[c].wait()
        for b in range(BSZ):
            pltpu.async_copy(
                buf,
                out_hbm.at[pl.ds(b * SEQ_LEN + pos0 + c * POS_CHUNK,
                                 POS_CHUNK)],
                ssem)
        if c + 2 < NUM_CHUNKS:
            drain_scatters(c)
            ghs.append(stage_in(c + 2))
    drain_scatters(NUM_CHUNKS - 2)
    drain_scatters(NUM_CHUNKS - 1)
    for h in tok_copies:
        h.wait()

    def any_true(mask_i32):
        m = mask_i32
        for sh in (1, 2, 4, 8):
            m = m | _lane_shuffle(m, lane ^ sh)
        return m[0] > 0

    toks = [
        inp_v[pl.ds(b * POS_PER_WORKER + v * LANES, LANES)]
        for b in range(BSZ) for v in range(VREGS_PER_SLICE)
    ]
    worker_acc = toks[0] == 0
    for t in toks[1:]:
        worker_acc = worker_acc | (t == 0)

    @pl.when(any_true(jnp.where(worker_acc, 1, 0)))
    def _fix_worker():
        for b in range(BSZ):
            for v in range(VREGS_PER_SLICE):
                tok = inp_v[pl.ds(b * POS_PER_WORKER + v * LANES, LANES)]

                @pl.when(any_true(jnp.where(tok == 0, 1, 0)))
                def _fix(tok=tok, b=b, v=v):
                    fix_idx[...] = jnp.where(
                        tok == 0, 0, lane + (pos0 + v * LANES + 1))
                    pltpu.async_copy(w_hbm.at[fix_idx], fixbuf, fsem).wait()
                    pltpu.sync_copy(
                        fixbuf,
                        out_hbm.at[pl.ds(
                            b * SEQ_LEN + pos0 + v * LANES, LANES)])


@jax.jit
def _sc_embed(inp_flat, weights):
    mesh = plsc.VectorSubcoreMesh(core_axis_name="c", subcore_axis_name="s")
    k = pl.kernel(
        _sc_body,
        out_type=jax.ShapeDtypeStruct((TOTAL_ROWS, EMBED_DIM), jnp.float32),
        mesh=mesh,
        scratch_types=[
            pltpu.VMEM((BSZ * POS_PER_WORKER,), jnp.int32),
            pltpu.VMEM((POS_PER_WORKER,), jnp.int32),
            pltpu.VMEM((LANES,), jnp.int32),
            pltpu.VMEM((LANES, EMBED_DIM), jnp.float32),
            pltpu.VMEM((POS_CHUNK, EMBED_DIM), jnp.float32),
            pltpu.VMEM((POS_CHUNK, EMBED_DIM), jnp.float32),
            pltpu.SemaphoreType.DMA,
            pltpu.SemaphoreType.DMA,
            pltpu.SemaphoreType.DMA,
            pltpu.SemaphoreType.DMA,
        ],
    )
    return k(inp_flat, weights)


def kernel(input, weights):
    inp_flat = input.reshape(-1)
    out = _sc_embed(inp_flat, weights)
    return out.reshape(BSZ, SEQ_LEN, EMBED_DIM)


# speedup vs baseline: 3.2052x; 1.0113x over previous
import jax
import jax.numpy as jnp
from jax import lax
from jax.experimental import pallas as pl
from jax.experimental.pallas import tpu as pltpu
from jax.experimental.pallas import tpu_sc as plsc

NUM_CORES = 2
NUM_SUBCORES = 16
LANES = 16
NUM_WORKERS = NUM_CORES * NUM_SUBCORES

BSZ = 4
SEQ_LEN = 4096
EMBED_DIM = 1024
TOTAL_ROWS = BSZ * SEQ_LEN
POS_PER_WORKER = SEQ_LEN // NUM_WORKERS
POS_CHUNK = 32
NUM_CHUNKS = POS_PER_WORKER // POS_CHUNK
VREGS_PER_SLICE = POS_PER_WORKER // LANES


def _lane_shuffle(x, idx):
    return lax.gather(
        x, idx[:, None],
        lax.GatherDimensionNumbers(
            offset_dims=(), collapsed_slice_dims=(0,), start_index_map=(0,)),
        (1,), mode=lax.GatherScatterMode.PROMISE_IN_BOUNDS)


def _sc_body(inp_hbm, w_hbm, out_hbm, inp_v, idx_v, fix_idx, fixbuf,
             rows0, rows1, gsem, ssem0, ssem1, fsem):
    wid = lax.axis_index("s") * NUM_CORES + lax.axis_index("c")
    pos0 = wid * POS_PER_WORKER

    bufs = (rows0, rows1)
    ssems = (ssem0, ssem1)
    lane = lax.iota(jnp.int32, LANES)

    for v in range(POS_PER_WORKER // LANES):
        idx_v[pl.ds(v * LANES, LANES)] = lane + (pos0 + v * LANES + 1)

    tok_copies = [
        pltpu.async_copy(
            inp_hbm.at[b, pl.ds(pos0, POS_PER_WORKER)],
            inp_v.at[pl.ds(b * POS_PER_WORKER, POS_PER_WORKER)], fsem)
        for b in range(BSZ)
    ]

    def drain_scatters(c):
        for _ in range(BSZ):
            pltpu.make_async_copy(
                out_hbm.at[pl.ds(0, POS_CHUNK)],
                bufs[c % 2],
                ssems[c % 2]).wait()

    def stage_in(c):
        return pltpu.async_copy(
            w_hbm.at[idx_v.at[pl.ds(c * POS_CHUNK, POS_CHUNK)]],
            bufs[c % 2], gsem)

    ghs = [stage_in(0), stage_in(1)]
    for c in range(NUM_CHUNKS):
        buf = bufs[c % 2]
        ssem = ssems[c % 2]
        ghs[c].wait()
        for b in range(BSZ):
            pltpu.async_copy(
                buf,
                out_hbm.at[pl.ds(b * SEQ_LEN + pos0 + c * POS_CHUNK,
                                 POS_CHUNK)],
                ssem)
        if c + 2 < NUM_CHUNKS:
            drain_scatters(c)
            ghs.append(stage_in(c + 2))
    drain_scatters(NUM_CHUNKS - 2)
    drain_scatters(NUM_CHUNKS - 1)
    for h in tok_copies:
        h.wait()

    acc = jnp.zeros((LANES,), jnp.int32)
    for g in range(BSZ * VREGS_PER_SLICE):
        tok = inp_v[pl.ds(g * LANES, LANES)]
        acc = acc | jnp.where(tok == 0, jnp.left_shift(jnp.int32(1), g), 0)
    for sh in (1, 2, 4, 8):
        acc = acc | _lane_shuffle(acc, lane ^ sh)
    groups_mask = acc[0]

    @pl.when(groups_mask != 0)
    def _fix_worker():
        for b in range(BSZ):
            for v in range(VREGS_PER_SLICE):
                g = b * VREGS_PER_SLICE + v

                @pl.when((groups_mask & jnp.left_shift(jnp.int32(1), g)) != 0)
                def _fix(b=b, v=v):
                    tok = inp_v[pl.ds(b * POS_PER_WORKER + v * LANES, LANES)]
                    fix_idx[...] = jnp.where(
                        tok == 0, 0, lane + (pos0 + v * LANES + 1))
                    pltpu.async_copy(w_hbm.at[fix_idx], fixbuf, fsem).wait()
                    pltpu.sync_copy(
                        fixbuf,
                        out_hbm.at[pl.ds(
                            b * SEQ_LEN + pos0 + v * LANES, LANES)])


@jax.jit
def _sc_embed(inp, weights):
    mesh = plsc.VectorSubcoreMesh(core_axis_name="c", subcore_axis_name="s")
    k = pl.kernel(
        _sc_body,
        out_type=jax.ShapeDtypeStruct((TOTAL_ROWS, EMBED_DIM), jnp.float32),
        mesh=mesh,
        scratch_types=[
            pltpu.VMEM((BSZ * POS_PER_WORKER,), jnp.int32),
            pltpu.VMEM((POS_PER_WORKER,), jnp.int32),
            pltpu.VMEM((LANES,), jnp.int32),
            pltpu.VMEM((LANES, EMBED_DIM), jnp.float32),
            pltpu.VMEM((POS_CHUNK, EMBED_DIM), jnp.float32),
            pltpu.VMEM((POS_CHUNK, EMBED_DIM), jnp.float32),
            pltpu.SemaphoreType.DMA,
            pltpu.SemaphoreType.DMA,
            pltpu.SemaphoreType.DMA,
            pltpu.SemaphoreType.DMA,
        ],
    )
    return k(inp, weights)


def kernel(input, weights):
    out = _sc_embed(input, weights)
    return out.reshape(BSZ, SEQ_LEN, EMBED_DIM)


# speedup vs baseline: 3.2525x; 1.0148x over previous
import jax
import jax.numpy as jnp
from jax import lax
from jax.experimental import pallas as pl
from jax.experimental.pallas import tpu as pltpu
from jax.experimental.pallas import tpu_sc as plsc

NUM_CORES = 2
NUM_SUBCORES = 16
LANES = 16
NUM_WORKERS = NUM_CORES * NUM_SUBCORES

BSZ = 4
SEQ_LEN = 4096
EMBED_DIM = 1024
TOTAL_ROWS = BSZ * SEQ_LEN
POS_PER_WORKER = SEQ_LEN // NUM_WORKERS
POS_CHUNK = 32
NUM_CHUNKS = POS_PER_WORKER // POS_CHUNK
VREGS_PER_SLICE = POS_PER_WORKER // LANES


def _lane_shuffle(x, idx):
    return lax.gather(
        x, idx[:, None],
        lax.GatherDimensionNumbers(
            offset_dims=(), collapsed_slice_dims=(0,), start_index_map=(0,)),
        (1,), mode=lax.GatherScatterMode.PROMISE_IN_BOUNDS)


NUM_BUFS = 3


def _sc_body(inp_hbm, w_hbm, out_hbm, inp_v, idx_v, fix_idx, fixbuf,
             rows0, rows1, rows2, gsem, ssem0, ssem1, ssem2, fsem):
    wid = lax.axis_index("s") * NUM_CORES + lax.axis_index("c")
    pos0 = wid * POS_PER_WORKER

    bufs = (rows0, rows1, rows2)
    ssems = (ssem0, ssem1, ssem2)
    lane = lax.iota(jnp.int32, LANES)

    for v in range(POS_PER_WORKER // LANES):
        idx_v[pl.ds(v * LANES, LANES)] = lane + (pos0 + v * LANES + 1)

    tok_copies = [
        pltpu.async_copy(
            inp_hbm.at[b, pl.ds(pos0, POS_PER_WORKER)],
            inp_v.at[pl.ds(b * POS_PER_WORKER, POS_PER_WORKER)], fsem)
        for b in range(BSZ)
    ]

    def drain_scatters(c):
        for _ in range(BSZ):
            pltpu.make_async_copy(
                out_hbm.at[pl.ds(0, POS_CHUNK)],
                bufs[c % NUM_BUFS],
                ssems[c % NUM_BUFS]).wait()

    def stage_in(c):
        return pltpu.async_copy(
            w_hbm.at[idx_v.at[pl.ds(c * POS_CHUNK, POS_CHUNK)]],
            bufs[c % NUM_BUFS], gsem)

    ghs = [stage_in(c) for c in range(NUM_BUFS)]
    for c in range(NUM_CHUNKS):
        buf = bufs[c % NUM_BUFS]
        ssem = ssems[c % NUM_BUFS]
        ghs[c].wait()
        for b in range(BSZ):
            pltpu.async_copy(
                buf,
                out_hbm.at[pl.ds(b * SEQ_LEN + pos0 + c * POS_CHUNK,
                                 POS_CHUNK)],
                ssem)
        if c + NUM_BUFS < NUM_CHUNKS:
            drain_scatters(c)
            ghs.append(stage_in(c + NUM_BUFS))
    for c in range(max(0, NUM_CHUNKS - NUM_BUFS), NUM_CHUNKS):
        drain_scatters(c)
    for h in tok_copies:
        h.wait()

    acc = jnp.zeros((LANES,), jnp.int32)
    for g in range(BSZ * VREGS_PER_SLICE):
        tok = inp_v[pl.ds(g * LANES, LANES)]
        acc = acc | jnp.where(tok == 0, jnp.left_shift(jnp.int32(1), g), 0)
    for sh in (1, 2, 4, 8):
        acc = acc | _lane_shuffle(acc, lane ^ sh)
    groups_mask = acc[0]

    @pl.when(groups_mask != 0)
    def _fix_worker():
        for b in range(BSZ):
            for v in range(VREGS_PER_SLICE):
                g = b * VREGS_PER_SLICE + v

                @pl.when((groups_mask & jnp.left_shift(jnp.int32(1), g)) != 0)
                def _fix(b=b, v=v):
                    tok = inp_v[pl.ds(b * POS_PER_WORKER + v * LANES, LANES)]
                    fix_idx[...] = jnp.where(
                        tok == 0, 0, lane + (pos0 + v * LANES + 1))
                    pltpu.async_copy(w_hbm.at[fix_idx], fixbuf, fsem).wait()
                    pltpu.sync_copy(
                        fixbuf,
                        out_hbm.at[pl.ds(
                            b * SEQ_LEN + pos0 + v * LANES, LANES)])


@jax.jit
def _sc_embed(inp, weights):
    mesh = plsc.VectorSubcoreMesh(core_axis_name="c", subcore_axis_name="s")
    k = pl.kernel(
        _sc_body,
        out_type=jax.ShapeDtypeStruct((TOTAL_ROWS, EMBED_DIM), jnp.float32),
        mesh=mesh,
        scratch_types=[
            pltpu.VMEM((BSZ * POS_PER_WORKER,), jnp.int32),
            pltpu.VMEM((POS_PER_WORKER,), jnp.int32),
            pltpu.VMEM((LANES,), jnp.int32),
            pltpu.VMEM((LANES, EMBED_DIM), jnp.float32),
            pltpu.VMEM((POS_CHUNK, EMBED_DIM), jnp.float32),
            pltpu.VMEM((POS_CHUNK, EMBED_DIM), jnp.float32),
            pltpu.VMEM((POS_CHUNK, EMBED_DIM), jnp.float32),
            pltpu.SemaphoreType.DMA,
            pltpu.SemaphoreType.DMA,
            pltpu.SemaphoreType.DMA,
            pltpu.SemaphoreType.DMA,
            pltpu.SemaphoreType.DMA,
        ],
    )
    return k(inp, weights)


def kernel(input, weights):
    out = _sc_embed(input, weights)
    return out.reshape(BSZ, SEQ_LEN, EMBED_DIM)


# speedup vs baseline: 3.2675x; 1.0046x over previous
import jax
import jax.numpy as jnp
from jax import lax
from jax.experimental import pallas as pl
from jax.experimental.pallas import tpu as pltpu
from jax.experimental.pallas import tpu_sc as plsc

NUM_CORES = 2
NUM_SUBCORES = 16
LANES = 16
NUM_WORKERS = NUM_CORES * NUM_SUBCORES

BSZ = 4
SEQ_LEN = 4096
EMBED_DIM = 1024
TOTAL_ROWS = BSZ * SEQ_LEN
POS_PER_WORKER = SEQ_LEN // NUM_WORKERS
POS_CHUNK = 32
NUM_CHUNKS = POS_PER_WORKER // POS_CHUNK
VREGS_PER_SLICE = POS_PER_WORKER // LANES


def _lane_shuffle(x, idx):
    return lax.gather(
        x, idx[:, None],
        lax.GatherDimensionNumbers(
            offset_dims=(), collapsed_slice_dims=(0,), start_index_map=(0,)),
        (1,), mode=lax.GatherScatterMode.PROMISE_IN_BOUNDS)


NUM_BUFS = 2
BUF_ROWS = 48
CHUNKS = ((0, 48), (48, 48), (96, 32))


def _sc_body(inp_hbm, w_hbm, out_hbm, inp_v, idx_v, fix_idx, fixbuf,
             rows0, rows1, gsem, ssem0, ssem1, fsem):
    wid = lax.axis_index("s") * NUM_CORES + lax.axis_index("c")
    pos0 = wid * POS_PER_WORKER

    bufs = (rows0, rows1)
    ssems = (ssem0, ssem1)
    lane = lax.iota(jnp.int32, LANES)

    for v in range(POS_PER_WORKER // LANES):
        idx_v[pl.ds(v * LANES, LANES)] = lane + (pos0 + v * LANES + 1)

    tok_copies = [
        pltpu.async_copy(
            inp_hbm.at[b, pl.ds(pos0, POS_PER_WORKER)],
            inp_v.at[pl.ds(b * POS_PER_WORKER, POS_PER_WORKER)], fsem)
        for b in range(BSZ)
    ]

    def drain_scatters(c):
        n = CHUNKS[c][1]
        for _ in range(BSZ):
            pltpu.make_async_copy(
                out_hbm.at[pl.ds(0, n)],
                bufs[c % NUM_BUFS].at[pl.ds(0, n)],
                ssems[c % NUM_BUFS]).wait()

    def stage_in(c):
        start, n = CHUNKS[c]
        return pltpu.async_copy(
            w_hbm.at[idx_v.at[pl.ds(start, n)]],
            bufs[c % NUM_BUFS].at[pl.ds(0, n)], gsem)

    ghs = [stage_in(c) for c in range(NUM_BUFS)]
    for c in range(len(CHUNKS)):
        start, n = CHUNKS[c]
        buf = bufs[c % NUM_BUFS]
        ssem = ssems[c % NUM_BUFS]
        ghs[c].wait()
        for b in range(BSZ):
            pltpu.async_copy(
                buf.at[pl.ds(0, n)],
                out_hbm.at[pl.ds(b * SEQ_LEN + pos0 + start, n)],
                ssem)
        if c + NUM_BUFS < len(CHUNKS):
            drain_scatters(c)
            ghs.append(stage_in(c + NUM_BUFS))
    for c in range(max(0, len(CHUNKS) - NUM_BUFS), len(CHUNKS)):
        drain_scatters(c)
    for h in tok_copies:
        h.wait()

    acc = jnp.zeros((LANES,), jnp.int32)
    for g in range(BSZ * VREGS_PER_SLICE):
        tok = inp_v[pl.ds(g * LANES, LANES)]
        acc = acc | jnp.where(tok == 0, jnp.left_shift(jnp.int32(1), g), 0)
    for sh in (1, 2, 4, 8):
        acc = acc | _lane_shuffle(acc, lane ^ sh)
    groups_mask = acc[0]

    @pl.when(groups_mask != 0)
    def _fix_worker():
        for b in range(BSZ):
            for v in range(VREGS_PER_SLICE):
                g = b * VREGS_PER_SLICE + v

                @pl.when((groups_mask & jnp.left_shift(jnp.int32(1), g)) != 0)
                def _fix(b=b, v=v):
                    tok = inp_v[pl.ds(b * POS_PER_WORKER + v * LANES, LANES)]
                    fix_idx[...] = jnp.where(
                        tok == 0, 0, lane + (pos0 + v * LANES + 1))
                    pltpu.async_copy(w_hbm.at[fix_idx], fixbuf, fsem).wait()
                    pltpu.sync_copy(
                        fixbuf,
                        out_hbm.at[pl.ds(
                            b * SEQ_LEN + pos0 + v * LANES, LANES)])


@jax.jit
def _sc_embed(inp, weights):
    mesh = plsc.VectorSubcoreMesh(core_axis_name="c", subcore_axis_name="s")
    k = pl.kernel(
        _sc_body,
        out_type=jax.ShapeDtypeStruct((TOTAL_ROWS, EMBED_DIM), jnp.float32),
        mesh=mesh,
        scratch_types=[
            pltpu.VMEM((BSZ * POS_PER_WORKER,), jnp.int32),
            pltpu.VMEM((POS_PER_WORKER,), jnp.int32),
            pltpu.VMEM((LANES,), jnp.int32),
            pltpu.VMEM((LANES, EMBED_DIM), jnp.float32),
            pltpu.VMEM((BUF_ROWS, EMBED_DIM), jnp.float32),
            pltpu.VMEM((BUF_ROWS, EMBED_DIM), jnp.float32),
            pltpu.SemaphoreType.DMA,
            pltpu.SemaphoreType.DMA,
            pltpu.SemaphoreType.DMA,
            pltpu.SemaphoreType.DMA,
        ],
    )
    return k(inp, weights)


def kernel(input, weights):
    out = _sc_embed(input, weights)
    return out.reshape(BSZ, SEQ_LEN, EMBED_DIM)


# speedup vs baseline: 3.2856x; 1.0055x over previous
import jax
import jax.numpy as jnp
from jax import lax
from jax.experimental import pallas as pl
from jax.experimental.pallas import tpu as pltpu
from jax.experimental.pallas import tpu_sc as plsc

NUM_CORES = 2
NUM_SUBCORES = 16
LANES = 16
NUM_WORKERS = NUM_CORES * NUM_SUBCORES

BSZ = 4
SEQ_LEN = 4096
EMBED_DIM = 1024
TOTAL_ROWS = BSZ * SEQ_LEN
POS_PER_WORKER = SEQ_LEN // NUM_WORKERS
VREGS_PER_SLICE = POS_PER_WORKER // LANES


def _lane_shuffle(x, idx):
    return lax.gather(
        x, idx[:, None],
        lax.GatherDimensionNumbers(
            offset_dims=(), collapsed_slice_dims=(0,), start_index_map=(0,)),
        (1,), mode=lax.GatherScatterMode.PROMISE_IN_BOUNDS)


NUM_BUFS = 2
BUF_ROWS = 48
CHUNKS = ((0, 48), (48, 48), (96, 32))


def _sc_body(inp_hbm, w_hbm, out_hbm, inp_v, idx_v, fix_idx, fixbuf,
             rows0, rows1, gsem, ssem0, ssem1, fsem):
    wid = lax.axis_index("s") * NUM_CORES + lax.axis_index("c")
    pos0 = wid * POS_PER_WORKER

    bufs = (rows0, rows1)
    ssems = (ssem0, ssem1)
    lane = lax.iota(jnp.int32, LANES)

    for v in range(POS_PER_WORKER // LANES):
        idx_v[pl.ds(v * LANES, LANES)] = lane + (pos0 + v * LANES + 1)

    tok_copies = [
        pltpu.async_copy(
            inp_hbm.at[b, pl.ds(pos0, POS_PER_WORKER)],
            inp_v.at[pl.ds(b * POS_PER_WORKER, POS_PER_WORKER)], fsem)
        for b in range(BSZ)
    ]

    def drain_scatters(c):
        n = CHUNKS[c][1]
        for _ in range(BSZ):
            pltpu.make_async_copy(
                out_hbm.at[pl.ds(0, n)],
                bufs[c % NUM_BUFS].at[pl.ds(0, n)],
                ssems[c % NUM_BUFS]).wait()

    def stage_in(c):
        start, n = CHUNKS[c]
        return pltpu.async_copy(
            w_hbm.at[idx_v.at[pl.ds(start, n)]],
            bufs[c % NUM_BUFS].at[pl.ds(0, n)], gsem)

    ghs = [stage_in(c) for c in range(NUM_BUFS)]
    for c in range(len(CHUNKS)):
        start, n = CHUNKS[c]
        buf = bufs[c % NUM_BUFS]
        ssem = ssems[c % NUM_BUFS]
        ghs[c].wait()
        for b in range(BSZ):
            pltpu.async_copy(
                buf.at[pl.ds(0, n)],
                out_hbm.at[pl.ds(b * SEQ_LEN + pos0 + start, n)],
                ssem)
        if c + NUM_BUFS < len(CHUNKS):
            drain_scatters(c)
            ghs.append(stage_in(c + NUM_BUFS))
    for c in range(max(0, len(CHUNKS) - NUM_BUFS), len(CHUNKS)):
        drain_scatters(c)
    for h in tok_copies:
        h.wait()

    acc = jnp.zeros((LANES,), jnp.int32)
    for g in range(BSZ * VREGS_PER_SLICE):
        tok = inp_v[pl.ds(g * LANES, LANES)]
        acc = acc | jnp.where(tok == 0, jnp.left_shift(jnp.int32(1), g), 0)
    for sh in (1, 2, 4, 8):
        acc = acc | _lane_shuffle(acc, lane ^ sh)
    groups_mask = acc[0]

    @pl.when(groups_mask != 0)
    def _fix_worker():
        for b in range(BSZ):
            for v in range(VREGS_PER_SLICE):
                g = b * VREGS_PER_SLICE + v

                @pl.when((groups_mask & jnp.left_shift(jnp.int32(1), g)) != 0)
                def _fix(b=b, v=v):
                    tok = inp_v[pl.ds(b * POS_PER_WORKER + v * LANES, LANES)]
                    fix_idx[...] = jnp.where(
                        tok == 0, 0, lane + (pos0 + v * LANES + 1))
                    pltpu.async_copy(w_hbm.at[fix_idx], fixbuf, fsem).wait()
                    pltpu.sync_copy(
                        fixbuf,
                        out_hbm.at[pl.ds(
                            b * SEQ_LEN + pos0 + v * LANES, LANES)])


@jax.jit
def _sc_embed(inp, weights):
    mesh = plsc.VectorSubcoreMesh(core_axis_name="c", subcore_axis_name="s")
    k = pl.kernel(
        _sc_body,
        out_type=jax.ShapeDtypeStruct((TOTAL_ROWS, EMBED_DIM), jnp.float32),
        mesh=mesh,
        scratch_types=[
            pltpu.VMEM((BSZ * POS_PER_WORKER,), jnp.int32),
            pltpu.VMEM((POS_PER_WORKER,), jnp.int32),
            pltpu.VMEM((LANES,), jnp.int32),
            pltpu.VMEM((LANES, EMBED_DIM), jnp.float32),
            pltpu.VMEM((BUF_ROWS, EMBED_DIM), jnp.float32),
            pltpu.VMEM((BUF_ROWS, EMBED_DIM), jnp.float32),
            pltpu.SemaphoreType.DMA,
            pltpu.SemaphoreType.DMA,
            pltpu.SemaphoreType.DMA,
            pltpu.SemaphoreType.DMA,
        ],
    )
    return k(inp, weights)


def kernel(input, weights):
    out = _sc_embed(input, weights)
    return out.reshape(BSZ, SEQ_LEN, EMBED_DIM)
